# baseline (speedup 1.0000x reference)
import functools

import jax
import jax.numpy as jnp
from jax import lax
from jax.experimental import pallas as pl
from jax.experimental.pallas import tpu as pltpu
from jax.experimental.pallas import tpu_sc as plsc

OLD_VOCAB = 100000
NEW_VOCAB = 1000
DIM = 64
PDIM = 128
BATCH = 4096
SEQ = 200

TOK = BATCH * SEQ
NUM_CORES = 2
NUM_SUBCORES = 16
NW = NUM_CORES * NUM_SUBCORES
PER_W = TOK // NW
CHUNK = 640
NCHUNK = PER_W // CHUNK
NPAIR = NCHUNK // 2

_mesh = plsc.VectorSubcoreMesh(core_axis_name="c", subcore_axis_name="s")


@functools.partial(
    pl.kernel,
    mesh=_mesh,
    out_type=jax.ShapeDtypeStruct((TOK, PDIM), jnp.float32),
    scratch_types=[
        pltpu.VMEM((PER_W,), jnp.int32),
        pltpu.VMEM((CHUNK, DIM), jnp.float32),
        pltpu.VMEM((CHUNK, DIM), jnp.float32),
        pltpu.SemaphoreType.DMA,
        pltpu.SemaphoreType.DMA,
        pltpu.SemaphoreType.DMA,
        pltpu.SemaphoreType.DMA,
    ],
    compiler_params=pltpu.CompilerParams(use_tc_tiling_on_sc=False),
)
def _gather_kernel(ids_hbm, table_hbm, out_hbm, idx_all, rows0, rows1,
                   sg0, sg1, ss0, ss1):
    wid = lax.axis_index("s") * NUM_CORES + lax.axis_index("c")
    base = wid * PER_W

    pltpu.sync_copy(ids_hbm.at[pl.ds(base, PER_W)], idx_all)

    rows = (rows0, rows1)
    sg = (sg0, sg1)
    ss = (ss0, ss1)

    def g_desc(c, b):
        return pltpu.make_async_copy(
            table_hbm.at[idx_all.at[pl.ds(c * CHUNK, CHUNK)]], rows[b], sg[b])

    def s_desc(c, b):
        return pltpu.make_async_copy(
            rows[b],
            out_hbm.at[pl.ds(base + c * CHUNK, CHUNK), pl.ds(0, DIM)], ss[b])

    def start_g(c, b):
        g_desc(c, b).start()

    def start_s(c, b):
        s_desc(c, b).start()

    def wait_g(b):
        g_desc(0, b).wait()

    def wait_s(b):
        s_desc(0, b).wait()

    start_g(0, 0)
    wait_g(0)
    start_s(0, 0)
    start_g(1, 1)

    def body(j, carry):
        wait_g(1)
        start_s(2 * j - 1, 1)
        wait_s(0)
        start_g(2 * j, 0)
        wait_g(0)
        start_s(2 * j, 0)
        wait_s(1)
        start_g(2 * j + 1, 1)
        return carry

    lax.fori_loop(1, NPAIR, body, 0)

    wait_g(1)
    start_s(NCHUNK - 1, 1)
    wait_s(0)
    wait_s(1)


_TCB = 64


def _tc_slice_body(x_ref, o_ref):
    o_ref[...] = x_ref[:, :, :DIM]


_tc_slice = pl.pallas_call(
    _tc_slice_body,
    grid=(BATCH // _TCB,),
    in_specs=[pl.BlockSpec((_TCB, SEQ, PDIM), lambda i: (i, 0, 0))],
    out_specs=pl.BlockSpec((_TCB, SEQ, DIM), lambda i: (i, 0, 0)),
    out_shape=jax.ShapeDtypeStruct((BATCH, SEQ, DIM), jnp.float32),
)


def kernel(input_ids, old_table, new_table):
    table = jnp.concatenate([old_table, new_table], axis=0)
    ids = input_ids.reshape(-1).astype(jnp.int32)
    out = _gather_kernel(ids, table)
    return _tc_slice(out.reshape(BATCH, SEQ, PDIM))


# speedup vs baseline: 1.8449x; 1.8449x over previous
import functools

import jax
import jax.numpy as jnp
from jax import lax
from jax.experimental import pallas as pl
from jax.experimental.pallas import tpu as pltpu
from jax.experimental.pallas import tpu_sc as plsc

OLD_VOCAB = 100000
NEW_VOCAB = 1000
DIM = 64
PDIM = 128
BATCH = 4096
SEQ = 200

TOK = BATCH * SEQ
NUM_CORES = 2
NUM_SUBCORES = 16
NW = NUM_CORES * NUM_SUBCORES
PER_W = TOK // NW
CHUNK = 640
NCHUNK = PER_W // CHUNK
NPAIR = NCHUNK // 2

_mesh = plsc.VectorSubcoreMesh(core_axis_name="c", subcore_axis_name="s")


@functools.partial(
    pl.kernel,
    mesh=_mesh,
    out_type=jax.ShapeDtypeStruct((TOK, PDIM), jnp.float32),
    scratch_types=[
        pltpu.VMEM((PER_W,), jnp.int32),
        pltpu.VMEM((CHUNK, DIM), jnp.float32),
        pltpu.VMEM((CHUNK, DIM), jnp.float32),
        pltpu.SemaphoreType.DMA,
        pltpu.SemaphoreType.DMA,
        pltpu.SemaphoreType.DMA,
        pltpu.SemaphoreType.DMA,
    ],
    compiler_params=pltpu.CompilerParams(use_tc_tiling_on_sc=False),
)
def _gather_kernel(ids_hbm, table_hbm, out_hbm, idx_all, rows0, rows1,
                   sg0, sg1, ss0, ss1):
    wid = lax.axis_index("s") * NUM_CORES + lax.axis_index("c")
    base = wid * PER_W

    pltpu.sync_copy(ids_hbm.at[pl.ds(base, PER_W)], idx_all)

    rows = (rows0, rows1)
    sg = (sg0, sg1)
    ss = (ss0, ss1)

    def g_desc(c, b):
        return pltpu.make_async_copy(
            table_hbm.at[idx_all.at[pl.ds(c * CHUNK, CHUNK)]], rows[b], sg[b])

    def s_desc(c, b):
        return pltpu.make_async_copy(
            rows[b],
            out_hbm.at[pl.ds(base + c * CHUNK, CHUNK), pl.ds(0, DIM)], ss[b])

    def start_g(c, b):
        g_desc(c, b).start()

    def start_s(c, b):
        s_desc(c, b).start()

    def wait_g(b):
        g_desc(0, b).wait()

    def wait_s(b):
        s_desc(0, b).wait()

    start_g(0, 0)
    wait_g(0)
    start_s(0, 0)
    start_g(1, 1)

    def body(j, carry):
        wait_g(1)
        start_s(2 * j - 1, 1)
        wait_s(0)
        start_g(2 * j, 0)
        wait_g(0)
        start_s(2 * j, 0)
        wait_s(1)
        start_g(2 * j + 1, 1)
        return carry

    lax.fori_loop(1, NPAIR, body, 0)

    wait_g(1)
    start_s(NCHUNK - 1, 1)
    wait_s(0)
    wait_s(1)


def kernel(input_ids, old_table, new_table):
    table = jnp.concatenate([old_table, new_table], axis=0)
    ids = input_ids.reshape(-1).astype(jnp.int32)
    out = _gather_kernel(ids, table)
    return out.reshape(BATCH, SEQ, PDIM)[:, :, :DIM]
